# baseline (speedup 1.0000x reference)
import functools

import jax
import jax.numpy as jnp
from jax import lax
from jax.experimental import pallas as pl
from jax.experimental.pallas import tpu as pltpu
from jax.experimental.pallas import tpu_sc as plsc

N = 100000
E = 1600000
G = 20000

NC = 2
NS = 16

N_TC = 102400
E_PAD = 1638400
G_PAD = 20480
BR = 2048
NBLK_TC = N_TC // BR

EB = 2048

_mesh = plsc.VectorSubcoreMesh(core_axis_name="c", subcore_axis_name="s")


def _deg_body(dst2d, z1, out, dbuf, ones128, acc, ssem):
    c = lax.axis_index("c")
    s = lax.axis_index("s")
    w = c * NS + s
    for k in range(8):
        ones128[pl.ds(k * 16, 16)] = jnp.full((16,), 1.0, jnp.float32)
    pltpu.sync_copy(z1, acc.at[pl.ds(s * 6400, 6400)])
    plsc.subcore_barrier()

    def block(b, _):
        r0 = w * 400 + b * 16
        pltpu.sync_copy(dst2d.at[pl.ds(r0, 16), :], dbuf)
        hs = []
        for j in range(16):
            hs.append(pltpu.async_copy(ones128, acc.at[dbuf.at[j]], ssem,
                                       add=True))
        for h in hs:
            h.wait()
        return 0

    lax.fori_loop(0, 25, block, 0)
    plsc.subcore_barrier()
    pltpu.sync_copy(acc.at[pl.ds(s * 6400, 6400)],
                    out.at[pl.ds(c * N_TC + s * 6400, 6400)])


def _sc_degree(dst2d, z1):
    return pl.kernel(
        _deg_body,
        out_type=jax.ShapeDtypeStruct((2 * N_TC,), jnp.float32),
        mesh=_mesh,
        scratch_types=[
            pltpu.VMEM((16, 128), jnp.int32),
            pltpu.VMEM((128,), jnp.float32),
            pltpu.VMEM_SHARED((N_TC,), jnp.float32),
            pltpu.SemaphoreType.DMA,
        ],
    )(dst2d, z1)


CHUNK = 50000
ACC_ROWS = 50176
ST = ACC_ROWS // NS
LAST = CHUNK - (NS - 1) * ST


def _agg_body(y, srcp, dstp, z2d, out,
              sbuf, dbuf, ig, isc, rows, acc, gsem, ssem):
    c = lax.axis_index("c")
    s = lax.axis_index("s")
    lo = c * CHUNK
    pltpu.sync_copy(z2d, acc.at[pl.ds(s * ST, ST), :])
    plsc.subcore_barrier()

    def block(b, _):
        e0 = s * (E_PAD // NS) + b * EB
        pltpu.sync_copy(srcp.at[pl.ds(e0, EB)], sbuf)
        pltpu.sync_copy(dstp.at[pl.ds(e0, EB)], dbuf)

        def batch(bi, _):
            for j in range(8):
                sv = sbuf[pl.ds(bi * 128 + j * 16, 16)]
                dv = dbuf[pl.ds(bi * 128 + j * 16, 16)]
                m = (dv >= lo) & (dv < lo + CHUNK)
                ig[pl.ds(j * 16, 16)] = jnp.where(m, sv, 0)
                isc[0, pl.ds(j * 16, 16)] = jnp.where(m, dv - lo, CHUNK)
            pltpu.async_copy(y.at[ig], rows, gsem).wait()
            pltpu.async_copy(rows, acc.at[isc.at[0]], ssem, add=True).wait()
            return 0

        lax.fori_loop(0, EB // 128, batch, 0)
        return 0

    lax.fori_loop(0, E_PAD // NS // EB, block, 0)

    plsc.subcore_barrier()
    @pl.when(s < NS - 1)
    def _():
        pltpu.sync_copy(acc.at[pl.ds(s * ST, ST), :],
                        out.at[pl.ds(lo + s * ST, ST), :])

    @pl.when(s == NS - 1)
    def _():
        pltpu.sync_copy(acc.at[pl.ds((NS - 1) * ST, LAST), :],
                        out.at[pl.ds(lo + (NS - 1) * ST, LAST), :])


def _sc_agg(y, srcp, dstp, z2d):
    return pl.kernel(
        _agg_body,
        out_type=jax.ShapeDtypeStruct((N_TC, 32), jnp.float32),
        mesh=_mesh,
        compiler_params=pltpu.CompilerParams(use_tc_tiling_on_sc=False),
        scratch_types=[
            pltpu.VMEM((EB,), jnp.int32),
            pltpu.VMEM((EB,), jnp.int32),
            pltpu.VMEM((128,), jnp.int32),
            pltpu.VMEM((1, 128), jnp.int32),
            pltpu.VMEM((128, 32), jnp.float32),
            pltpu.VMEM_SHARED((ACC_ROWS, 32), jnp.float32),
            pltpu.SemaphoreType.DMA,
            pltpu.SemaphoreType.DMA,
        ],
    )(y, srcp, dstp, z2d)


def _pool_body(h3, batch2d, z2d, z1, sums_out, cnt_out,
               rbuf, ibuf, ones128, sacc, cacc, ssem):
    c = lax.axis_index("c")
    s = lax.axis_index("s")
    for k in range(8):
        ones128[pl.ds(k * 16, 16)] = jnp.full((16,), 1.0, jnp.float32)
    pltpu.sync_copy(z2d.at[pl.ds(0, 1280), :], sacc.at[pl.ds(s * 1280, 1280), :])
    pltpu.sync_copy(z1.at[pl.ds(0, 1280)], cacc.at[pl.ds(s * 1280, 1280)])
    plsc.subcore_barrier()

    def block(b, _):
        base = c * (N_TC // 2) + s * 3200 + b * 128
        pltpu.sync_copy(h3.at[pl.ds(base, 128), :], rbuf)
        pltpu.sync_copy(batch2d.at[pl.ds(base // 128, 1), :], ibuf)
        pltpu.sync_copy(rbuf, sacc.at[ibuf.at[0]], add=True)
        pltpu.sync_copy(ones128, cacc.at[ibuf.at[0]], add=True)
        return 0

    lax.fori_loop(0, 25, block, 0)
    plsc.subcore_barrier()
    pltpu.sync_copy(sacc.at[pl.ds(s * 1280, 1280), :],
                    sums_out.at[pl.ds(c * G_PAD + s * 1280, 1280), :])
    pltpu.sync_copy(cacc.at[pl.ds(s * 1280, 1280)],
                    cnt_out.at[pl.ds(c * G_PAD + s * 1280, 1280)])


def _sc_pool(h3, batch2d, z2d, z1):
    return pl.kernel(
        _pool_body,
        out_type=[jax.ShapeDtypeStruct((2 * G_PAD, 64), jnp.float32),
                  jax.ShapeDtypeStruct((2 * G_PAD,), jnp.float32)],
        mesh=_mesh,
        compiler_params=pltpu.CompilerParams(use_tc_tiling_on_sc=False),
        scratch_types=[
            pltpu.VMEM((128, 64), jnp.float32),
            pltpu.VMEM((1, 128), jnp.int32),
            pltpu.VMEM((128,), jnp.float32),
            pltpu.VMEM_SHARED((G_PAD + 16, 64), jnp.float32),
            pltpu.VMEM_SHARED((G_PAD + 16,), jnp.float32),
            pltpu.SemaphoreType.DMA,
        ],
    )(h3, batch2d, z2d, z1)


def _tc_a_body(x_ref, w_ref, da_ref, db_ref, y_ref, dinv_ref):
    deg = da_ref[...] + db_ref[...] + 1.0
    dv = lax.rsqrt(deg)
    xw = jnp.dot(x_ref[...], w_ref[...], preferred_element_type=jnp.float32)
    y_ref[...] = xw * dv
    dinv_ref[...] = dv


def _tc_a(x8, W1p, degA, degB):
    return pl.pallas_call(
        _tc_a_body,
        grid=(NBLK_TC,),
        in_specs=[
            pl.BlockSpec((BR, 8), lambda i: (i, 0)),
            pl.BlockSpec((8, 32), lambda i: (0, 0)),
            pl.BlockSpec((BR, 1), lambda i: (i, 0)),
            pl.BlockSpec((BR, 1), lambda i: (i, 0)),
        ],
        out_specs=[pl.BlockSpec((BR, 32), lambda i: (i, 0)),
                   pl.BlockSpec((BR, 1), lambda i: (i, 0))],
        out_shape=[jax.ShapeDtypeStruct((N_TC, 32), jnp.float32),
                   jax.ShapeDtypeStruct((N_TC, 1), jnp.float32)],
    )(x8, W1p, degA, degB)


def _tc_mid_body(agg_ref, y_ref, dinv_ref, b_ref, t_ref, st_ref, acc_ref):
    i = pl.program_id(0)
    t = dinv_ref[...] * (agg_ref[...] + y_ref[...]) + b_ref[...]
    t_ref[...] = t
    base = i * BR
    rid = lax.broadcasted_iota(jnp.int32, (BR, 1), 0) + base
    ts = jnp.where(rid < N, t, 0.0)
    s1 = jnp.sum(ts, axis=0, keepdims=True)
    s2 = jnp.sum(ts * ts, axis=0, keepdims=True)
    ps = jnp.concatenate([s1, s2], axis=0)

    @pl.when(i == 0)
    def _():
        acc_ref[...] = jnp.zeros_like(acc_ref)

    acc_ref[...] += ps

    @pl.when(i == NBLK_TC - 1)
    def _():
        st_ref[...] = acc_ref[...]


def _tc_mid(agg, y, dinv, b, F):
    return pl.pallas_call(
        _tc_mid_body,
        grid=(NBLK_TC,),
        in_specs=[
            pl.BlockSpec((BR, F), lambda i: (i, 0)),
            pl.BlockSpec((BR, F), lambda i: (i, 0)),
            pl.BlockSpec((BR, 1), lambda i: (i, 0)),
            pl.BlockSpec((F,), lambda i: (0,)),
        ],
        out_specs=[pl.BlockSpec((BR, F), lambda i: (i, 0)),
                   pl.BlockSpec((2, F), lambda i: (0, 0))],
        out_shape=[jax.ShapeDtypeStruct((N_TC, F), jnp.float32),
                   jax.ShapeDtypeStruct((2, F), jnp.float32)],
        scratch_shapes=[pltpu.VMEM((2, F), jnp.float32)],
    )(agg, y, dinv, b)


def _tc_post_body(t_ref, st_ref, g_ref, bt_ref, dinv_ref, w_ref, y_ref):
    m = st_ref[0:1, :] * (1.0 / N)
    v = st_ref[1:2, :] * (1.0 / N) - m * m
    scale = g_ref[...] * lax.rsqrt(v + 1e-5)
    h = jnp.maximum((t_ref[...] - m) * scale + bt_ref[...], 0.0)
    y_ref[...] = jnp.dot(h, w_ref[...],
                         preferred_element_type=jnp.float32) * dinv_ref[...]


def _tc_post(t, st, g, bt, dinv, Wn, F, Fn):
    return pl.pallas_call(
        _tc_post_body,
        grid=(NBLK_TC,),
        in_specs=[
            pl.BlockSpec((BR, F), lambda i: (i, 0)),
            pl.BlockSpec((2, F), lambda i: (0, 0)),
            pl.BlockSpec((1, F), lambda i: (0, 0)),
            pl.BlockSpec((1, F), lambda i: (0, 0)),
            pl.BlockSpec((BR, 1), lambda i: (i, 0)),
            pl.BlockSpec((F, Fn), lambda i: (0, 0)),
        ],
        out_specs=pl.BlockSpec((BR, Fn), lambda i: (i, 0)),
        out_shape=jax.ShapeDtypeStruct((N_TC, Fn), jnp.float32),
    )(t, st, g.reshape(1, F), bt.reshape(1, F), dinv, Wn)


def _tc_h3_body(agg_ref, y_ref, dinv_ref, b_ref, h_ref):
    h_ref[...] = jnp.maximum(
        dinv_ref[...] * (agg_ref[...] + y_ref[...]) + b_ref[...], 0.0)


def _tc_h3(agg, y, dinv, b):
    return pl.pallas_call(
        _tc_h3_body,
        grid=(NBLK_TC,),
        in_specs=[
            pl.BlockSpec((BR, 64), lambda i: (i, 0)),
            pl.BlockSpec((BR, 64), lambda i: (i, 0)),
            pl.BlockSpec((BR, 1), lambda i: (i, 0)),
            pl.BlockSpec((64,), lambda i: (0,)),
        ],
        out_specs=pl.BlockSpec((BR, 64), lambda i: (i, 0)),
        out_shape=jax.ShapeDtypeStruct((N_TC, 64), jnp.float32),
    )(agg, y, dinv, b)


def _tc_head_body(sa_ref, sb_ref, ca_ref, cb_ref, w1_ref, b1_ref, w2_ref,
                  b2_ref, o_ref):
    ssum = sa_ref[...] + sb_ref[...]
    cnt = ca_ref[...] + cb_ref[...]
    pooled = ssum / jnp.maximum(cnt, 1.0)
    h = jnp.maximum(jnp.dot(pooled, w1_ref[...],
                            preferred_element_type=jnp.float32) + b1_ref[...],
                    0.0)
    o_ref[...] = jnp.dot(h, w2_ref[...],
                         preferred_element_type=jnp.float32) + b2_ref[...]


def _tc_head(sa, sb, ca, cb, fw1, fb1, fw2p, fb2p):
    BG = 2048
    return pl.pallas_call(
        _tc_head_body,
        grid=(G_PAD // BG,),
        in_specs=[
            pl.BlockSpec((BG, 64), lambda i: (i, 0)),
            pl.BlockSpec((BG, 64), lambda i: (i, 0)),
            pl.BlockSpec((BG, 1), lambda i: (i, 0)),
            pl.BlockSpec((BG, 1), lambda i: (i, 0)),
            pl.BlockSpec((64, 32), lambda i: (0, 0)),
            pl.BlockSpec((32,), lambda i: (0,)),
            pl.BlockSpec((32, 128), lambda i: (0, 0)),
            pl.BlockSpec((128,), lambda i: (0,)),
        ],
        out_specs=pl.BlockSpec((BG, 128), lambda i: (i, 0)),
        out_shape=jax.ShapeDtypeStruct((G_PAD, 128), jnp.float32),
    )(sa, sb, ca, cb, fw1, fb1, fw2p, fb2p)


def kernel(x, edge_index, batch, W1, b1, g1, bt1, W2, b2, g2, bt2, W3, b3,
           fw1, fb1, fw2, fb2):
    f32 = jnp.float32
    src_p = jnp.concatenate([edge_index[0],
                             jnp.zeros((E_PAD - E,), jnp.int32)])
    dst_p = jnp.concatenate([edge_index[1],
                             jnp.full((E_PAD - E,), N, jnp.int32)])
    dst2d = dst_p.reshape(E_PAD // 128, 128)
    batch2d = jnp.concatenate([batch, jnp.full((N_TC - N,), G, jnp.int32)]
                              ).reshape(N_TC // 128, 128)
    x8 = jnp.zeros((N_TC, 8), f32).at[:N, :5].set(x)
    W1p = jnp.zeros((8, 32), f32).at[:5, :].set(W1)
    fw2p = jnp.zeros((32, 128), f32).at[:, :2].set(fw2)
    fb2p = jnp.zeros((128,), f32).at[:2].set(fb2)
    z1 = jnp.zeros((6400,), f32)
    z32 = jnp.zeros((3136, 32), f32)
    z64 = jnp.zeros((1568, 64), f32)

    deg1d = _sc_degree(dst2d, z1)
    degA = deg1d[:N_TC].reshape(N_TC, 1)
    degB = deg1d[N_TC:].reshape(N_TC, 1)

    y1, dinv = _tc_a(x8, W1p, degA, degB)

    agg1 = _sc_agg(y1, src_p, dst_p, z32)
    t1, st1 = _tc_mid(agg1, y1, dinv, b1, 32)
    y2 = _tc_post(t1, st1, g1, bt1, dinv, W2, 32, 64)

    agg2 = jnp.concatenate(
        [_sc_agg(y2[:, :32], src_p, dst_p, z32),
         _sc_agg(y2[:, 32:], src_p, dst_p, z32)], axis=1)
    t2, st2 = _tc_mid(agg2, y2, dinv, b2, 64)
    y3 = _tc_post(t2, st2, g2, bt2, dinv, W3, 64, 64)

    agg3 = jnp.concatenate(
        [_sc_agg(y3[:, :32], src_p, dst_p, z32),
         _sc_agg(y3[:, 32:], src_p, dst_p, z32)], axis=1)
    h3 = _tc_h3(agg3, y3, dinv, b3)

    sums, cnt = _sc_pool(h3, batch2d, z64, z1)
    sa = sums[:G_PAD]
    sb = sums[G_PAD:]
    ca = cnt[:G_PAD].reshape(G_PAD, 1)
    cb = cnt[G_PAD:].reshape(G_PAD, 1)

    out = _tc_head(sa, sb, ca, cb, fw1, fb1, fw2p, fb2p)
    return out[:G, :2]


# speedup vs baseline: 1.0033x; 1.0033x over previous
import functools

import jax
import jax.numpy as jnp
from jax import lax
from jax.experimental import pallas as pl
from jax.experimental.pallas import tpu as pltpu
from jax.experimental.pallas import tpu_sc as plsc

N = 100000
E = 1600000
G = 20000

NC = 2
NS = 16

N_TC = 102400
E_PAD = 1638400
G_PAD = 20480
BR = 2048
NBLK_TC = N_TC // BR

EB = 2048

_mesh = plsc.VectorSubcoreMesh(core_axis_name="c", subcore_axis_name="s")


def _deg_body(dst2d, z1, out, dbuf, ones128, acc, ssem):
    c = lax.axis_index("c")
    s = lax.axis_index("s")
    w = c * NS + s
    for k in range(8):
        ones128[pl.ds(k * 16, 16)] = jnp.full((16,), 1.0, jnp.float32)
    pltpu.sync_copy(z1, acc.at[pl.ds(s * 6400, 6400)])
    plsc.subcore_barrier()

    def block(b, _):
        r0 = w * 400 + b * 16
        pltpu.sync_copy(dst2d.at[pl.ds(r0, 16), :], dbuf)
        hs = []
        for j in range(16):
            hs.append(pltpu.async_copy(ones128, acc.at[dbuf.at[j]], ssem,
                                       add=True))
        for h in hs:
            h.wait()
        return 0

    lax.fori_loop(0, 25, block, 0)
    plsc.subcore_barrier()
    pltpu.sync_copy(acc.at[pl.ds(s * 6400, 6400)],
                    out.at[pl.ds(c * N_TC + s * 6400, 6400)])


def _sc_degree(dst2d, z1):
    return pl.kernel(
        _deg_body,
        out_type=jax.ShapeDtypeStruct((2 * N_TC,), jnp.float32),
        mesh=_mesh,
        scratch_types=[
            pltpu.VMEM((16, 128), jnp.int32),
            pltpu.VMEM((128,), jnp.float32),
            pltpu.VMEM_SHARED((N_TC,), jnp.float32),
            pltpu.SemaphoreType.DMA,
        ],
    )(dst2d, z1)


CHUNK = 50000
ACC_ROWS = 50176
ST = ACC_ROWS // NS
LAST = CHUNK - (NS - 1) * ST


NSLOT = 4
SB = 128
NSB = EB // SB
NBLKS = E_PAD // NS // EB


def _agg_body(y, srcp, dstp, z2d, out,
              sbuf, dbuf, ig, isc, rows, acc,
              g0, g1, g2, g3, s0, s1, s2, s3):
    gsems = [g0, g1, g2, g3]
    ssems = [s0, s1, s2, s3]
    c = lax.axis_index("c")
    s = lax.axis_index("s")
    lo = c * CHUNK
    pltpu.sync_copy(z2d, acc.at[pl.ds(s * ST, ST), :])
    plsc.subcore_barrier()

    def wait_scatter(k):
        pltpu.make_async_copy(y.at[pl.ds(0, SB), :],
                              acc.at[pl.ds(0, SB), :], ssems[k]).wait()

    def wait_gather(k):
        pltpu.make_async_copy(y.at[pl.ds(0, SB), :],
                              rows.at[k], gsems[k]).wait()

    def fire_gather(j):
        pltpu.async_copy(y.at[ig.at[pl.ds(j * SB, SB)]],
                         rows.at[j % NSLOT], gsems[j % NSLOT])

    def fire_scatter(j):
        pltpu.async_copy(rows.at[j % NSLOT],
                         acc.at[isc.at[pl.ds(j * SB, SB)]],
                         ssems[j % NSLOT], add=True)

    def block(b, _):
        e0 = s * (E_PAD // NS) + b * EB
        pltpu.sync_copy(srcp.at[pl.ds(e0, EB)], sbuf)
        pltpu.sync_copy(dstp.at[pl.ds(e0, EB)], dbuf)
        for j in range(NSB):
            k = j % NSLOT
            if j >= NSLOT:
                wait_scatter(k)
            else:
                @pl.when(b >= 1)
                def _():
                    wait_scatter(k)
            for i in range(SB // 16):
                o = j * SB + i * 16
                sv = sbuf[pl.ds(o, 16)]
                dv = dbuf[pl.ds(o, 16)]
                m = (dv >= lo) & (dv < lo + CHUNK)
                ig[pl.ds(o, 16)] = jnp.where(m, sv, 0)
                isc[pl.ds(o, 16)] = jnp.where(m, dv - lo, CHUNK)
            fire_gather(j)
            kp = (j - 1) % NSLOT
            if j >= 1:
                wait_gather(kp)
                fire_scatter(j - 1)
            else:
                @pl.when(b >= 1)
                def _():
                    wait_gather(kp)
                    fire_scatter(NSB - 1)
        return 0

    lax.fori_loop(0, NBLKS, block, 0)
    wait_gather((NSB - 1) % NSLOT)
    fire_scatter(NSB - 1)
    for k in range(NSLOT):
        wait_scatter(k)

    plsc.subcore_barrier()
    @pl.when(s < NS - 1)
    def _():
        pltpu.sync_copy(acc.at[pl.ds(s * ST, ST), :],
                        out.at[pl.ds(lo + s * ST, ST), :])

    @pl.when(s == NS - 1)
    def _():
        pltpu.sync_copy(acc.at[pl.ds((NS - 1) * ST, LAST), :],
                        out.at[pl.ds(lo + (NS - 1) * ST, LAST), :])


def _sc_agg(y, srcp, dstp, z2d):
    return pl.kernel(
        _agg_body,
        out_type=jax.ShapeDtypeStruct((N_TC, 32), jnp.float32),
        mesh=_mesh,
        compiler_params=pltpu.CompilerParams(use_tc_tiling_on_sc=False),
        scratch_types=[
            pltpu.VMEM((EB,), jnp.int32),
            pltpu.VMEM((EB,), jnp.int32),
            pltpu.VMEM((EB,), jnp.int32),
            pltpu.VMEM((EB,), jnp.int32),
            pltpu.VMEM((NSLOT, SB, 32), jnp.float32),
            pltpu.VMEM_SHARED((ACC_ROWS, 32), jnp.float32),
        ] + [pltpu.SemaphoreType.DMA] * 8,
    )(y, srcp, dstp, z2d)


def _pool_body(h3, batch2d, z2d, z1, sums_out, cnt_out,
               rbuf, ibuf, ones128, sacc, cacc, ssem):
    c = lax.axis_index("c")
    s = lax.axis_index("s")
    for k in range(8):
        ones128[pl.ds(k * 16, 16)] = jnp.full((16,), 1.0, jnp.float32)
    pltpu.sync_copy(z2d.at[pl.ds(0, 1280), :], sacc.at[pl.ds(s * 1280, 1280), :])
    pltpu.sync_copy(z1.at[pl.ds(0, 1280)], cacc.at[pl.ds(s * 1280, 1280)])
    plsc.subcore_barrier()

    def block(b, _):
        base = c * (N_TC // 2) + s * 3200 + b * 128
        pltpu.sync_copy(h3.at[pl.ds(base, 128), :], rbuf)
        pltpu.sync_copy(batch2d.at[pl.ds(base // 128, 1), :], ibuf)
        pltpu.sync_copy(rbuf, sacc.at[ibuf.at[0]], add=True)
        pltpu.sync_copy(ones128, cacc.at[ibuf.at[0]], add=True)
        return 0

    lax.fori_loop(0, 25, block, 0)
    plsc.subcore_barrier()
    pltpu.sync_copy(sacc.at[pl.ds(s * 1280, 1280), :],
                    sums_out.at[pl.ds(c * G_PAD + s * 1280, 1280), :])
    pltpu.sync_copy(cacc.at[pl.ds(s * 1280, 1280)],
                    cnt_out.at[pl.ds(c * G_PAD + s * 1280, 1280)])


def _sc_pool(h3, batch2d, z2d, z1):
    return pl.kernel(
        _pool_body,
        out_type=[jax.ShapeDtypeStruct((2 * G_PAD, 64), jnp.float32),
                  jax.ShapeDtypeStruct((2 * G_PAD,), jnp.float32)],
        mesh=_mesh,
        compiler_params=pltpu.CompilerParams(use_tc_tiling_on_sc=False),
        scratch_types=[
            pltpu.VMEM((128, 64), jnp.float32),
            pltpu.VMEM((1, 128), jnp.int32),
            pltpu.VMEM((128,), jnp.float32),
            pltpu.VMEM_SHARED((G_PAD + 16, 64), jnp.float32),
            pltpu.VMEM_SHARED((G_PAD + 16,), jnp.float32),
            pltpu.SemaphoreType.DMA,
        ],
    )(h3, batch2d, z2d, z1)


def _tc_a_body(x_ref, w_ref, da_ref, db_ref, y_ref, dinv_ref):
    deg = da_ref[...] + db_ref[...] + 1.0
    dv = lax.rsqrt(deg)
    xw = jnp.dot(x_ref[...], w_ref[...], preferred_element_type=jnp.float32)
    y_ref[...] = xw * dv
    dinv_ref[...] = dv


def _tc_a(x8, W1p, degA, degB):
    return pl.pallas_call(
        _tc_a_body,
        grid=(NBLK_TC,),
        in_specs=[
            pl.BlockSpec((BR, 8), lambda i: (i, 0)),
            pl.BlockSpec((8, 32), lambda i: (0, 0)),
            pl.BlockSpec((BR, 1), lambda i: (i, 0)),
            pl.BlockSpec((BR, 1), lambda i: (i, 0)),
        ],
        out_specs=[pl.BlockSpec((BR, 32), lambda i: (i, 0)),
                   pl.BlockSpec((BR, 1), lambda i: (i, 0))],
        out_shape=[jax.ShapeDtypeStruct((N_TC, 32), jnp.float32),
                   jax.ShapeDtypeStruct((N_TC, 1), jnp.float32)],
    )(x8, W1p, degA, degB)


def _tc_mid_body(agg_ref, y_ref, dinv_ref, b_ref, t_ref, st_ref, acc_ref):
    i = pl.program_id(0)
    t = dinv_ref[...] * (agg_ref[...] + y_ref[...]) + b_ref[...]
    t_ref[...] = t
    base = i * BR
    rid = lax.broadcasted_iota(jnp.int32, (BR, 1), 0) + base
    ts = jnp.where(rid < N, t, 0.0)
    s1 = jnp.sum(ts, axis=0, keepdims=True)
    s2 = jnp.sum(ts * ts, axis=0, keepdims=True)
    ps = jnp.concatenate([s1, s2], axis=0)

    @pl.when(i == 0)
    def _():
        acc_ref[...] = jnp.zeros_like(acc_ref)

    acc_ref[...] += ps

    @pl.when(i == NBLK_TC - 1)
    def _():
        st_ref[...] = acc_ref[...]


def _tc_mid(agg, y, dinv, b, F):
    return pl.pallas_call(
        _tc_mid_body,
        grid=(NBLK_TC,),
        in_specs=[
            pl.BlockSpec((BR, F), lambda i: (i, 0)),
            pl.BlockSpec((BR, F), lambda i: (i, 0)),
            pl.BlockSpec((BR, 1), lambda i: (i, 0)),
            pl.BlockSpec((F,), lambda i: (0,)),
        ],
        out_specs=[pl.BlockSpec((BR, F), lambda i: (i, 0)),
                   pl.BlockSpec((2, F), lambda i: (0, 0))],
        out_shape=[jax.ShapeDtypeStruct((N_TC, F), jnp.float32),
                   jax.ShapeDtypeStruct((2, F), jnp.float32)],
        scratch_shapes=[pltpu.VMEM((2, F), jnp.float32)],
    )(agg, y, dinv, b)


def _tc_post_body(t_ref, st_ref, g_ref, bt_ref, dinv_ref, w_ref, y_ref):
    m = st_ref[0:1, :] * (1.0 / N)
    v = st_ref[1:2, :] * (1.0 / N) - m * m
    scale = g_ref[...] * lax.rsqrt(v + 1e-5)
    h = jnp.maximum((t_ref[...] - m) * scale + bt_ref[...], 0.0)
    y_ref[...] = jnp.dot(h, w_ref[...],
                         preferred_element_type=jnp.float32) * dinv_ref[...]


def _tc_post(t, st, g, bt, dinv, Wn, F, Fn):
    return pl.pallas_call(
        _tc_post_body,
        grid=(NBLK_TC,),
        in_specs=[
            pl.BlockSpec((BR, F), lambda i: (i, 0)),
            pl.BlockSpec((2, F), lambda i: (0, 0)),
            pl.BlockSpec((1, F), lambda i: (0, 0)),
            pl.BlockSpec((1, F), lambda i: (0, 0)),
            pl.BlockSpec((BR, 1), lambda i: (i, 0)),
            pl.BlockSpec((F, Fn), lambda i: (0, 0)),
        ],
        out_specs=pl.BlockSpec((BR, Fn), lambda i: (i, 0)),
        out_shape=jax.ShapeDtypeStruct((N_TC, Fn), jnp.float32),
    )(t, st, g.reshape(1, F), bt.reshape(1, F), dinv, Wn)


def _tc_h3_body(agg_ref, y_ref, dinv_ref, b_ref, h_ref):
    h_ref[...] = jnp.maximum(
        dinv_ref[...] * (agg_ref[...] + y_ref[...]) + b_ref[...], 0.0)


def _tc_h3(agg, y, dinv, b):
    return pl.pallas_call(
        _tc_h3_body,
        grid=(NBLK_TC,),
        in_specs=[
            pl.BlockSpec((BR, 64), lambda i: (i, 0)),
            pl.BlockSpec((BR, 64), lambda i: (i, 0)),
            pl.BlockSpec((BR, 1), lambda i: (i, 0)),
            pl.BlockSpec((64,), lambda i: (0,)),
        ],
        out_specs=pl.BlockSpec((BR, 64), lambda i: (i, 0)),
        out_shape=jax.ShapeDtypeStruct((N_TC, 64), jnp.float32),
    )(agg, y, dinv, b)


def _tc_head_body(sa_ref, sb_ref, ca_ref, cb_ref, w1_ref, b1_ref, w2_ref,
                  b2_ref, o_ref):
    ssum = sa_ref[...] + sb_ref[...]
    cnt = ca_ref[...] + cb_ref[...]
    pooled = ssum / jnp.maximum(cnt, 1.0)
    h = jnp.maximum(jnp.dot(pooled, w1_ref[...],
                            preferred_element_type=jnp.float32) + b1_ref[...],
                    0.0)
    o_ref[...] = jnp.dot(h, w2_ref[...],
                         preferred_element_type=jnp.float32) + b2_ref[...]


def _tc_head(sa, sb, ca, cb, fw1, fb1, fw2p, fb2p):
    BG = 2048
    return pl.pallas_call(
        _tc_head_body,
        grid=(G_PAD // BG,),
        in_specs=[
            pl.BlockSpec((BG, 64), lambda i: (i, 0)),
            pl.BlockSpec((BG, 64), lambda i: (i, 0)),
            pl.BlockSpec((BG, 1), lambda i: (i, 0)),
            pl.BlockSpec((BG, 1), lambda i: (i, 0)),
            pl.BlockSpec((64, 32), lambda i: (0, 0)),
            pl.BlockSpec((32,), lambda i: (0,)),
            pl.BlockSpec((32, 128), lambda i: (0, 0)),
            pl.BlockSpec((128,), lambda i: (0,)),
        ],
        out_specs=pl.BlockSpec((BG, 128), lambda i: (i, 0)),
        out_shape=jax.ShapeDtypeStruct((G_PAD, 128), jnp.float32),
    )(sa, sb, ca, cb, fw1, fb1, fw2p, fb2p)


def kernel(x, edge_index, batch, W1, b1, g1, bt1, W2, b2, g2, bt2, W3, b3,
           fw1, fb1, fw2, fb2):
    f32 = jnp.float32
    src_p = jnp.concatenate([edge_index[0],
                             jnp.zeros((E_PAD - E,), jnp.int32)])
    dst_p = jnp.concatenate([edge_index[1],
                             jnp.full((E_PAD - E,), N, jnp.int32)])
    dst2d = dst_p.reshape(E_PAD // 128, 128)
    batch2d = jnp.concatenate([batch, jnp.full((N_TC - N,), G, jnp.int32)]
                              ).reshape(N_TC // 128, 128)
    x8 = jnp.zeros((N_TC, 8), f32).at[:N, :5].set(x)
    W1p = jnp.zeros((8, 32), f32).at[:5, :].set(W1)
    fw2p = jnp.zeros((32, 128), f32).at[:, :2].set(fw2)
    fb2p = jnp.zeros((128,), f32).at[:2].set(fb2)
    z1 = jnp.zeros((6400,), f32)
    z32 = jnp.zeros((3136, 32), f32)
    z64 = jnp.zeros((1568, 64), f32)

    deg1d = _sc_degree(dst2d, z1)
    degA = deg1d[:N_TC].reshape(N_TC, 1)
    degB = deg1d[N_TC:].reshape(N_TC, 1)

    y1, dinv = _tc_a(x8, W1p, degA, degB)

    agg1 = _sc_agg(y1, src_p, dst_p, z32)
    t1, st1 = _tc_mid(agg1, y1, dinv, b1, 32)
    y2 = _tc_post(t1, st1, g1, bt1, dinv, W2, 32, 64)

    agg2 = jnp.concatenate(
        [_sc_agg(y2[:, :32], src_p, dst_p, z32),
         _sc_agg(y2[:, 32:], src_p, dst_p, z32)], axis=1)
    t2, st2 = _tc_mid(agg2, y2, dinv, b2, 64)
    y3 = _tc_post(t2, st2, g2, bt2, dinv, W3, 64, 64)

    agg3 = jnp.concatenate(
        [_sc_agg(y3[:, :32], src_p, dst_p, z32),
         _sc_agg(y3[:, 32:], src_p, dst_p, z32)], axis=1)
    h3 = _tc_h3(agg3, y3, dinv, b3)

    sums, cnt = _sc_pool(h3, batch2d, z64, z1)
    sa = sums[:G_PAD]
    sb = sums[G_PAD:]
    ca = cnt[:G_PAD].reshape(G_PAD, 1)
    cb = cnt[G_PAD:].reshape(G_PAD, 1)

    out = _tc_head(sa, sb, ca, cb, fw1, fb1, fw2p, fb2p)
    return out[:G, :2]


# speedup vs baseline: 2.4556x; 2.4474x over previous
import functools

import jax
import jax.numpy as jnp
from jax import lax
from jax.experimental import pallas as pl
from jax.experimental.pallas import tpu as pltpu
from jax.experimental.pallas import tpu_sc as plsc

N = 100000
E = 1600000
G = 20000

NC = 2
NS = 16

N_TC = 102400
E_PAD = 1638400
G_PAD = 20480
BR = 2048
NBLK_TC = N_TC // BR

EB = 2048
SB = 128

_mesh = plsc.VectorSubcoreMesh(core_axis_name="c", subcore_axis_name="s")


def _deg_body(dst2d, z1, out, dbuf, ones128, acc, ssem):
    c = lax.axis_index("c")
    s = lax.axis_index("s")
    w = c * NS + s
    for k in range(8):
        ones128[pl.ds(k * 16, 16)] = jnp.full((16,), 1.0, jnp.float32)
    pltpu.sync_copy(z1, acc.at[pl.ds(s * 6400, 6400)])
    plsc.subcore_barrier()

    def block(b, _):
        r0 = w * 400 + b * 16
        pltpu.sync_copy(dst2d.at[pl.ds(r0, 16), :], dbuf)
        hs = []
        for j in range(16):
            hs.append(pltpu.async_copy(ones128, acc.at[dbuf.at[j]], ssem,
                                       add=True))
        for h in hs:
            h.wait()
        return 0

    lax.fori_loop(0, 25, block, 0)
    plsc.subcore_barrier()
    pltpu.sync_copy(acc.at[pl.ds(s * 6400, 6400)],
                    out.at[pl.ds(c * N_TC + s * 6400, 6400)])


def _sc_degree(dst2d, z1):
    return pl.kernel(
        _deg_body,
        out_type=jax.ShapeDtypeStruct((2 * N_TC,), jnp.float32),
        mesh=_mesh,
        scratch_types=[
            pltpu.VMEM((16, 128), jnp.int32),
            pltpu.VMEM((128,), jnp.float32),
            pltpu.VMEM_SHARED((N_TC,), jnp.float32),
            pltpu.SemaphoreType.DMA,
        ],
    )(dst2d, z1)


CHUNK = 50000
ACC_ROWS = 50176
ST = ACC_ROWS // NS
LAST = CHUNK - (NS - 1) * ST
NSB = EB // SB
NBLKS = E_PAD // NS // EB


def _agg_body(y, srcp, dstp, z2d, out,
              sbuf, dbuf, ig, isc, rows, acc, gsem, ssem):
    c = lax.axis_index("c")
    s = lax.axis_index("s")
    lo = c * CHUNK
    pltpu.sync_copy(z2d, acc.at[pl.ds(s * ST, ST), :])
    plsc.subcore_barrier()

    def block(b, _):
        e0 = s * (E_PAD // NS) + b * EB
        pltpu.sync_copy(srcp.at[pl.ds(e0, EB)], sbuf)
        pltpu.sync_copy(dstp.at[pl.ds(e0, EB)], dbuf)
        for j in range(NSB):
            for i in range(SB // 16):
                o = j * SB + i * 16
                sv = sbuf[pl.ds(o, 16)]
                dv = dbuf[pl.ds(o, 16)]
                m = (dv >= lo) & (dv < lo + CHUNK)
                ig[pl.ds(o, 16)] = jnp.where(m, sv, 0)
                isc[pl.ds(o, 16)] = jnp.where(m, dv - lo, CHUNK)
            pltpu.async_copy(y.at[ig.at[pl.ds(j * SB, SB)]],
                             rows, gsem).wait()
            pltpu.async_copy(rows, acc.at[isc.at[pl.ds(j * SB, SB)]],
                             ssem, add=True).wait()
        return 0

    lax.fori_loop(0, NBLKS, block, 0)

    plsc.subcore_barrier()
    @pl.when(s < NS - 1)
    def _():
        pltpu.sync_copy(acc.at[pl.ds(s * ST, ST), :],
                        out.at[pl.ds(lo + s * ST, ST), :])

    @pl.when(s == NS - 1)
    def _():
        pltpu.sync_copy(acc.at[pl.ds((NS - 1) * ST, LAST), :],
                        out.at[pl.ds(lo + (NS - 1) * ST, LAST), :])


def _sc_agg(y, srcp, dstp, z2d, FW, DT):
    return pl.kernel(
        _agg_body,
        out_type=jax.ShapeDtypeStruct((N_TC, FW), DT),
        mesh=_mesh,
        compiler_params=pltpu.CompilerParams(use_tc_tiling_on_sc=False),
        scratch_types=[
            pltpu.VMEM((EB,), jnp.int32),
            pltpu.VMEM((EB,), jnp.int32),
            pltpu.VMEM((EB,), jnp.int32),
            pltpu.VMEM((EB,), jnp.int32),
            pltpu.VMEM((SB, FW), DT),
            pltpu.VMEM_SHARED((ACC_ROWS, FW), DT),
            pltpu.SemaphoreType.DMA,
            pltpu.SemaphoreType.DMA,
        ],
    )(y, srcp, dstp, z2d)


def _pool_body(h3, batch2d, z2d, z1, sums_out, cnt_out,
               rbuf, ibuf, ones128, sacc, cacc, ssem):
    c = lax.axis_index("c")
    s = lax.axis_index("s")
    for k in range(8):
        ones128[pl.ds(k * 16, 16)] = jnp.full((16,), 1.0, jnp.float32)
    pltpu.sync_copy(z2d.at[pl.ds(0, 1280), :], sacc.at[pl.ds(s * 1280, 1280), :])
    pltpu.sync_copy(z1.at[pl.ds(0, 1280)], cacc.at[pl.ds(s * 1280, 1280)])
    plsc.subcore_barrier()

    def block(b, _):
        base = c * (N_TC // 2) + s * 3200 + b * 128
        pltpu.sync_copy(h3.at[pl.ds(base, 128), :], rbuf)
        pltpu.sync_copy(batch2d.at[pl.ds(base // 128, 1), :], ibuf)
        pltpu.sync_copy(rbuf, sacc.at[ibuf.at[0]], add=True)
        pltpu.sync_copy(ones128, cacc.at[ibuf.at[0]], add=True)
        return 0

    lax.fori_loop(0, 25, block, 0)
    plsc.subcore_barrier()
    pltpu.sync_copy(sacc.at[pl.ds(s * 1280, 1280), :],
                    sums_out.at[pl.ds(c * G_PAD + s * 1280, 1280), :])
    pltpu.sync_copy(cacc.at[pl.ds(s * 1280, 1280)],
                    cnt_out.at[pl.ds(c * G_PAD + s * 1280, 1280)])


def _sc_pool(h3, batch2d, z2d, z1):
    return pl.kernel(
        _pool_body,
        out_type=[jax.ShapeDtypeStruct((2 * G_PAD, 64), jnp.float32),
                  jax.ShapeDtypeStruct((2 * G_PAD,), jnp.float32)],
        mesh=_mesh,
        compiler_params=pltpu.CompilerParams(use_tc_tiling_on_sc=False),
        scratch_types=[
            pltpu.VMEM((128, 64), jnp.float32),
            pltpu.VMEM((1, 128), jnp.int32),
            pltpu.VMEM((128,), jnp.float32),
            pltpu.VMEM_SHARED((G_PAD + 16, 64), jnp.float32),
            pltpu.VMEM_SHARED((G_PAD + 16,), jnp.float32),
            pltpu.SemaphoreType.DMA,
        ],
    )(h3, batch2d, z2d, z1)


def _tc_a_body(x_ref, da_ref, db_ref, xs_ref, dinv_ref):
    deg = da_ref[...] + db_ref[...] + 1.0
    dv = lax.rsqrt(deg)
    xs_ref[...] = x_ref[...] * dv
    dinv_ref[...] = dv


def _tc_a(x8, degA, degB):
    return pl.pallas_call(
        _tc_a_body,
        grid=(NBLK_TC,),
        in_specs=[
            pl.BlockSpec((BR, 8), lambda i: (i, 0)),
            pl.BlockSpec((BR, 1), lambda i: (i, 0)),
            pl.BlockSpec((BR, 1), lambda i: (i, 0)),
        ],
        out_specs=[pl.BlockSpec((BR, 8), lambda i: (i, 0)),
                   pl.BlockSpec((BR, 1), lambda i: (i, 0))],
        out_shape=[jax.ShapeDtypeStruct((N_TC, 8), jnp.float32),
                   jax.ShapeDtypeStruct((N_TC, 1), jnp.float32)],
    )(x8, degA, degB)


def _tc_mid_body(agg_ref, ys_ref, dinv_ref, w_ref, b_ref, t_ref, st_ref,
                 acc_ref):
    i = pl.program_id(0)
    u = dinv_ref[...] * (agg_ref[...].astype(jnp.float32) + ys_ref[...])
    t = jnp.dot(u, w_ref[...], preferred_element_type=jnp.float32) + b_ref[...]
    t_ref[...] = t
    base = i * BR
    rid = lax.broadcasted_iota(jnp.int32, (BR, 1), 0) + base
    ts = jnp.where(rid < N, t, 0.0)
    s1 = jnp.sum(ts, axis=0, keepdims=True)
    s2 = jnp.sum(ts * ts, axis=0, keepdims=True)
    ps = jnp.concatenate([s1, s2], axis=0)

    @pl.when(i == 0)
    def _():
        acc_ref[...] = jnp.zeros_like(acc_ref)

    acc_ref[...] += ps

    @pl.when(i == NBLK_TC - 1)
    def _():
        st_ref[...] = acc_ref[...]


def _tc_mid(agg, ys, dinv, W, b, Fi, Fo):
    return pl.pallas_call(
        _tc_mid_body,
        grid=(NBLK_TC,),
        in_specs=[
            pl.BlockSpec((BR, Fi), lambda i: (i, 0)),
            pl.BlockSpec((BR, Fi), lambda i: (i, 0)),
            pl.BlockSpec((BR, 1), lambda i: (i, 0)),
            pl.BlockSpec((Fi, Fo), lambda i: (0, 0)),
            pl.BlockSpec((Fo,), lambda i: (0,)),
        ],
        out_specs=[pl.BlockSpec((BR, Fo), lambda i: (i, 0)),
                   pl.BlockSpec((2, Fo), lambda i: (0, 0))],
        out_shape=[jax.ShapeDtypeStruct((N_TC, Fo), jnp.float32),
                   jax.ShapeDtypeStruct((2, Fo), jnp.float32)],
        scratch_shapes=[pltpu.VMEM((2, Fo), jnp.float32)],
    )(agg, ys, dinv, W, b)


def _tc_post_body(t_ref, st_ref, g_ref, bt_ref, dinv_ref, hf_ref, hb_ref):
    m = st_ref[0:1, :] * (1.0 / N)
    v = st_ref[1:2, :] * (1.0 / N) - m * m
    scale = g_ref[...] * lax.rsqrt(v + 1e-5)
    h = jnp.maximum((t_ref[...] - m) * scale + bt_ref[...], 0.0)
    hs = h * dinv_ref[...]
    hf_ref[...] = hs
    hb_ref[...] = hs.astype(jnp.bfloat16)


def _tc_post(t, st, g, bt, dinv, F):
    return pl.pallas_call(
        _tc_post_body,
        grid=(NBLK_TC,),
        in_specs=[
            pl.BlockSpec((BR, F), lambda i: (i, 0)),
            pl.BlockSpec((2, F), lambda i: (0, 0)),
            pl.BlockSpec((1, F), lambda i: (0, 0)),
            pl.BlockSpec((1, F), lambda i: (0, 0)),
            pl.BlockSpec((BR, 1), lambda i: (i, 0)),
        ],
        out_specs=[pl.BlockSpec((BR, F), lambda i: (i, 0)),
                   pl.BlockSpec((BR, F), lambda i: (i, 0))],
        out_shape=[jax.ShapeDtypeStruct((N_TC, F), jnp.float32),
                   jax.ShapeDtypeStruct((N_TC, F), jnp.bfloat16)],
    )(t, st, g.reshape(1, F), bt.reshape(1, F), dinv)


def _tc_h3_body(agg_ref, ys_ref, dinv_ref, w_ref, b_ref, h_ref):
    u = dinv_ref[...] * (agg_ref[...].astype(jnp.float32) + ys_ref[...])
    h_ref[...] = jnp.maximum(
        jnp.dot(u, w_ref[...], preferred_element_type=jnp.float32)
        + b_ref[...], 0.0)


def _tc_h3(agg, ys, dinv, W, b):
    return pl.pallas_call(
        _tc_h3_body,
        grid=(NBLK_TC,),
        in_specs=[
            pl.BlockSpec((BR, 64), lambda i: (i, 0)),
            pl.BlockSpec((BR, 64), lambda i: (i, 0)),
            pl.BlockSpec((BR, 1), lambda i: (i, 0)),
            pl.BlockSpec((64, 64), lambda i: (0, 0)),
            pl.BlockSpec((64,), lambda i: (0,)),
        ],
        out_specs=pl.BlockSpec((BR, 64), lambda i: (i, 0)),
        out_shape=jax.ShapeDtypeStruct((N_TC, 64), jnp.float32),
    )(agg, ys, dinv, W, b)


def _tc_head_body(sa_ref, sb_ref, ca_ref, cb_ref, w1_ref, b1_ref, w2_ref,
                  b2_ref, o_ref):
    ssum = sa_ref[...] + sb_ref[...]
    cnt = ca_ref[...] + cb_ref[...]
    pooled = ssum / jnp.maximum(cnt, 1.0)
    h = jnp.maximum(jnp.dot(pooled, w1_ref[...],
                            preferred_element_type=jnp.float32) + b1_ref[...],
                    0.0)
    o_ref[...] = jnp.dot(h, w2_ref[...],
                         preferred_element_type=jnp.float32) + b2_ref[...]


def _tc_head(sa, sb, ca, cb, fw1, fb1, fw2p, fb2p):
    BG = 2048
    return pl.pallas_call(
        _tc_head_body,
        grid=(G_PAD // BG,),
        in_specs=[
            pl.BlockSpec((BG, 64), lambda i: (i, 0)),
            pl.BlockSpec((BG, 64), lambda i: (i, 0)),
            pl.BlockSpec((BG, 1), lambda i: (i, 0)),
            pl.BlockSpec((BG, 1), lambda i: (i, 0)),
            pl.BlockSpec((64, 32), lambda i: (0, 0)),
            pl.BlockSpec((32,), lambda i: (0,)),
            pl.BlockSpec((32, 128), lambda i: (0, 0)),
            pl.BlockSpec((128,), lambda i: (0,)),
        ],
        out_specs=pl.BlockSpec((BG, 128), lambda i: (i, 0)),
        out_shape=jax.ShapeDtypeStruct((G_PAD, 128), jnp.float32),
    )(sa, sb, ca, cb, fw1, fb1, fw2p, fb2p)


def kernel(x, edge_index, batch, W1, b1, g1, bt1, W2, b2, g2, bt2, W3, b3,
           fw1, fb1, fw2, fb2):
    f32 = jnp.float32
    bf16 = jnp.bfloat16
    src_p = jnp.concatenate([edge_index[0],
                             jnp.zeros((E_PAD - E,), jnp.int32)])
    dst_p = jnp.concatenate([edge_index[1],
                             jnp.full((E_PAD - E,), N, jnp.int32)])
    dst2d = dst_p.reshape(E_PAD // 128, 128)
    batch2d = jnp.concatenate([batch, jnp.full((N_TC - N,), G, jnp.int32)]
                              ).reshape(N_TC // 128, 128)
    x8 = jnp.zeros((N_TC, 8), f32).at[:N, :5].set(x)
    W1p = jnp.zeros((8, 32), f32).at[:5, :].set(W1)
    fw2p = jnp.zeros((32, 128), f32).at[:, :2].set(fw2)
    fb2p = jnp.zeros((128,), f32).at[:2].set(fb2)
    z1 = jnp.zeros((6400,), f32)
    z8 = jnp.zeros((ST, 8), f32)
    zb32 = jnp.zeros((ST, 32), bf16)
    zb64 = jnp.zeros((ST, 64), bf16)
    z64 = jnp.zeros((1568, 64), f32)

    deg1d = _sc_degree(dst2d, z1)
    degA = deg1d[:N_TC].reshape(N_TC, 1)
    degB = deg1d[N_TC:].reshape(N_TC, 1)

    xs, dinv = _tc_a(x8, degA, degB)

    agg1 = _sc_agg(xs, src_p, dst_p, z8, 8, f32)
    t1, st1 = _tc_mid(agg1, xs, dinv, W1p, b1, 8, 32)
    h1f, h1b = _tc_post(t1, st1, g1, bt1, dinv, 32)

    agg2 = _sc_agg(h1b, src_p, dst_p, zb32, 32, bf16)
    t2, st2 = _tc_mid(agg2, h1f, dinv, W2, b2, 32, 64)
    h2f, h2b = _tc_post(t2, st2, g2, bt2, dinv, 64)

    agg3 = _sc_agg(h2b, src_p, dst_p, zb64, 64, bf16)
    h3 = _tc_h3(agg3, h2f, dinv, W3, b3)

    sums, cnt = _sc_pool(h3, batch2d, z64, z1)
    sa = sums[:G_PAD]
    sb = sums[G_PAD:]
    ca = cnt[:G_PAD].reshape(G_PAD, 1)
    cb = cnt[G_PAD:].reshape(G_PAD, 1)

    out = _tc_head(sa, sb, ca, cb, fw1, fb1, fw2p, fb2p)
    return out[:G, :2]
